# baseline (speedup 1.0000x reference)
import functools

import jax
import jax.numpy as jnp
from jax import lax
from jax.experimental import pallas as pl
from jax.experimental.pallas import tpu as pltpu
from jax.experimental.pallas import tpu_sc as plsc

N = 10000
E = 320000
D = 128
H = 64
O = 2

NC = 2
NS = 16
NW = NC * NS
CHUNK = 128
EPT = 10240
NCH = EPT // CHUNK
NCHH = NCH // 2
E_PAD = NW * EPT
ROWS_IDX = E_PAD // CHUNK
NACC = 10240
ZR = NACC // NS
ZB = 40
W = 128

_mesh = plsc.VectorSubcoreMesh(core_axis_name="c", subcore_axis_name="s")


def _deg_body(dst_hbm, out_hbm, didx, ones_v, zbuf, acc, sem):
    cid = lax.axis_index("c")
    sid = lax.axis_index("s")
    w = cid * NS + sid
    pltpu.sync_copy(dst_hbm.at[pl.ds(w * NCH, NCH)], didx)
    for j in range(ZR // 16):
        zbuf[pl.ds(j * 16, 16)] = jnp.zeros((16,), jnp.float32)
    for j in range(CHUNK // 16):
        ones_v[pl.ds(j * 16, 16)] = jnp.ones((16,), jnp.float32)
    pltpu.sync_copy(zbuf, acc.at[pl.ds(sid * ZR, ZR)])
    plsc.subcore_barrier()

    def step(j, _):
        pltpu.sync_copy(ones_v, acc.at[didx.at[j]], add=True)
        return 0

    lax.fori_loop(0, NCH, step, 0)
    plsc.subcore_barrier()
    pltpu.sync_copy(acc.at[pl.ds(sid * ZR, ZR)],
                    out_hbm.at[pl.ds(cid * NACC + sid * ZR, ZR)])


@functools.partial(
    pl.kernel,
    out_type=jax.ShapeDtypeStruct((NC * NACC,), jnp.float32),
    mesh=_mesh,
    scratch_types=[
        pltpu.VMEM((NCH, CHUNK), jnp.int32),
        pltpu.VMEM((CHUNK,), jnp.float32),
        pltpu.VMEM((ZR,), jnp.float32),
        pltpu.VMEM_SHARED((NACC,), jnp.float32),
        pltpu.SemaphoreType.DMA,
    ],
)
def _deg_call(dst_hbm, out_hbm, didx, ones_v, zbuf, acc, sem):
    _deg_body(dst_hbm, out_hbm, didx, ones_v, zbuf, acc, sem)


def _scat_body(g_hbm, src_hbm, dst_hbm, out_hbm,
               sidx, didx, rows, zbuf, acc, gsem0, gsem1):
    cid = lax.axis_index("c")
    sid = lax.axis_index("s")
    w = cid * NS + sid
    for r in range(ZB):
        for c in range(W // 16):
            zbuf[r, pl.ds(c * 16, 16)] = jnp.zeros((16,), jnp.float32)

    def zstep(t, _):
        pltpu.sync_copy(zbuf, acc.at[pl.ds(sid * ZR + t * ZB, ZB)])
        return 0

    lax.fori_loop(0, ZR // ZB, zstep, 0)
    plsc.subcore_barrier()

    gsems = (gsem0, gsem1)
    for half in range(2):
        base = w * NCH + half * NCHH
        pltpu.sync_copy(src_hbm.at[pl.ds(base, NCHH)], sidx)
        pltpu.sync_copy(dst_hbm.at[pl.ds(base, NCHH)], didx)
        for b in range(2):
            pltpu.async_copy(g_hbm.at[sidx.at[b]], rows.at[b], gsems[b])

        def step(k, _):
            for b in range(2):
                j = 2 * k + b
                pltpu.make_async_copy(g_hbm.at[sidx.at[j]], rows.at[b],
                                      gsems[b]).wait()
                pltpu.sync_copy(rows.at[b], acc.at[didx.at[j]], add=True)
                pltpu.async_copy(g_hbm.at[sidx.at[j + 2]], rows.at[b],
                                 gsems[b])
            return 0

        lax.fori_loop(0, NCHH // 2 - 1, step, 0)
        for b in range(2):
            j = NCHH - 2 + b
            pltpu.make_async_copy(g_hbm.at[sidx.at[j]], rows.at[b],
                                  gsems[b]).wait()
            pltpu.sync_copy(rows.at[b], acc.at[didx.at[j]], add=True)

    plsc.subcore_barrier()
    pltpu.sync_copy(acc.at[pl.ds(sid * ZR, ZR)],
                    out_hbm.at[cid].at[pl.ds(sid * ZR, ZR)])


@functools.partial(
    pl.kernel,
    out_type=jax.ShapeDtypeStruct((NC, NACC, W), jnp.float32),
    mesh=_mesh,
    scratch_types=[
        pltpu.VMEM((NCHH, CHUNK), jnp.int32),
        pltpu.VMEM((NCHH, CHUNK), jnp.int32),
        pltpu.VMEM((2, CHUNK, W), jnp.float32),
        pltpu.VMEM((ZB, W), jnp.float32),
        pltpu.VMEM_SHARED((NACC, W), jnp.float32),
        pltpu.SemaphoreType.DMA,
        pltpu.SemaphoreType.DMA,
    ],
)
def _scat_call(g_hbm, src_hbm, dst_hbm, out_hbm,
               sidx, didx, rows, zbuf, acc, gsem0, gsem1):
    _scat_body(g_hbm, src_hbm, dst_hbm, out_hbm,
               sidx, didx, rows, zbuf, acc, gsem0, gsem1)


def _tcA1_body(x_ref, w1_ref, h1_ref):
    h1_ref[...] = jnp.dot(x_ref[...], w1_ref[...],
                          preferred_element_type=jnp.float32)


def _tcA2_body(deg_ref, h1_ref, g1_ref, dinv_ref):
    deg = (deg_ref[0, :N] + deg_ref[1, :N] + 1.0)[:, None]
    dinv = lax.rsqrt(deg)
    g1_ref[...] = jnp.concatenate(
        [h1_ref[...] * dinv, jnp.zeros((N, W - H), jnp.float32)], axis=1)
    dinv_ref[...] = dinv


def _tcB_body(s_ref, g_ref, dinv_ref, b_ref, w_ref, out_ref):
    dinv = dinv_ref[...]
    a = ((s_ref[0, :N, :H] + s_ref[1, :N, :H] + g_ref[:, :H]) * dinv
         + b_ref[...])
    h = jnp.maximum(a, 0.0)
    g2 = jnp.dot(h, w_ref[...], preferred_element_type=jnp.float32) * dinv
    out_ref[...] = jnp.concatenate(
        [g2, jnp.zeros((N, W - H), jnp.float32)], axis=1)


def _tcC_body(s_ref, g_ref, dinv_ref, b_ref, w_ref, blin_ref, out_ref):
    a = ((s_ref[0, :N, :H] + s_ref[1, :N, :H] + g_ref[:, :H]) * dinv_ref[...]
         + b_ref[...])
    h = jnp.maximum(a, 0.0)
    out_ref[...] = jnp.dot(h, w_ref[...],
                           preferred_element_type=jnp.float32) + blin_ref[...]


def kernel(x, edge_index, W1, b1, W2, b2, Wlin, blin):
    src = edge_index[0]
    dst = edge_index[1]
    pad = E_PAD - E
    ppos = jnp.arange(pad, dtype=jnp.int32)
    src_pad = (ppos * 131) % N
    dst_pad = N + (ppos % (NACC - N))
    src_p = jnp.concatenate([src, src_pad]).reshape(ROWS_IDX, CHUNK)
    dst_p = jnp.concatenate([dst, dst_pad]).reshape(ROWS_IDX, CHUNK)

    h1 = pl.pallas_call(
        _tcA1_body,
        out_shape=jax.ShapeDtypeStruct((N, H), jnp.float32),
    )(x, W1)
    deg_parts = _deg_call(dst_p).reshape(NC, NACC)

    g1, dinv = pl.pallas_call(
        _tcA2_body,
        out_shape=(jax.ShapeDtypeStruct((N, W), jnp.float32),
                   jax.ShapeDtypeStruct((N, 1), jnp.float32)),
    )(deg_parts, h1)

    s1 = _scat_call(g1, src_p, dst_p)

    g2 = pl.pallas_call(
        _tcB_body,
        out_shape=jax.ShapeDtypeStruct((N, W), jnp.float32),
    )(s1, g1, dinv, b1.reshape(1, H), W2)

    s2 = _scat_call(g2, src_p, dst_p)

    logits = pl.pallas_call(
        _tcC_body,
        out_shape=jax.ShapeDtypeStruct((N, O), jnp.float32),
    )(s2, g2, dinv, b2.reshape(1, H), Wlin, blin.reshape(1, O))
    return logits


# speedup vs baseline: 1.0225x; 1.0225x over previous
import functools

import jax
import jax.numpy as jnp
from jax import lax
from jax.experimental import pallas as pl
from jax.experimental.pallas import tpu as pltpu
from jax.experimental.pallas import tpu_sc as plsc

N = 10000
E = 320000
D = 128
H = 64
O = 2

NC = 2
NS = 16
NW = NC * NS
CHUNK = 64
EPT = 10240
NCH = EPT // CHUNK
NCHQ = NCH // 4
E_PAD = NW * EPT
ROWS_IDX = E_PAD // CHUNK
NACC = 10240
ZR = NACC // NS
ZB = 40
W = 128

_mesh = plsc.VectorSubcoreMesh(core_axis_name="c", subcore_axis_name="s")


def _deg_body(dst_hbm, out_hbm, didx, ones_v, zbuf, acc, sem):
    cid = lax.axis_index("c")
    sid = lax.axis_index("s")
    w = cid * NS + sid
    pltpu.sync_copy(dst_hbm.at[pl.ds(w * NCH, NCH)], didx)
    for j in range(ZR // 16):
        zbuf[pl.ds(j * 16, 16)] = jnp.zeros((16,), jnp.float32)
    for j in range(CHUNK // 16):
        ones_v[pl.ds(j * 16, 16)] = jnp.ones((16,), jnp.float32)
    pltpu.sync_copy(zbuf, acc.at[pl.ds(sid * ZR, ZR)])
    plsc.subcore_barrier()

    def step(j, _):
        pltpu.sync_copy(ones_v, acc.at[didx.at[j]], add=True)
        return 0

    lax.fori_loop(0, NCH, step, 0)
    plsc.subcore_barrier()
    pltpu.sync_copy(acc.at[pl.ds(sid * ZR, ZR)],
                    out_hbm.at[pl.ds(cid * NACC + sid * ZR, ZR)])


@functools.partial(
    pl.kernel,
    out_type=jax.ShapeDtypeStruct((NC * NACC,), jnp.float32),
    mesh=_mesh,
    scratch_types=[
        pltpu.VMEM((NCH, CHUNK), jnp.int32),
        pltpu.VMEM((CHUNK,), jnp.float32),
        pltpu.VMEM((ZR,), jnp.float32),
        pltpu.VMEM_SHARED((NACC,), jnp.float32),
        pltpu.SemaphoreType.DMA,
    ],
)
def _deg_call(dst_hbm, out_hbm, didx, ones_v, zbuf, acc, sem):
    _deg_body(dst_hbm, out_hbm, didx, ones_v, zbuf, acc, sem)


def _scat_body(g_hbm, src_hbm, dst_hbm, out_hbm,
               sidx, didx, rows, zbuf, acc, gsem0, gsem1, gsem2, gsem3):
    cid = lax.axis_index("c")
    sid = lax.axis_index("s")
    w = cid * NS + sid
    for r in range(ZB):
        for c in range(W // 16):
            zbuf[r, pl.ds(c * 16, 16)] = jnp.zeros((16,), jnp.float32)

    def zstep(t, _):
        pltpu.sync_copy(zbuf, acc.at[pl.ds(sid * ZR + t * ZB, ZB)])
        return 0

    lax.fori_loop(0, ZR // ZB, zstep, 0)
    plsc.subcore_barrier()

    gsems = (gsem0, gsem1, gsem2, gsem3)
    for quarter in range(4):
        base = w * NCH + quarter * NCHQ
        pltpu.sync_copy(src_hbm.at[pl.ds(base, NCHQ)], sidx)
        pltpu.sync_copy(dst_hbm.at[pl.ds(base, NCHQ)], didx)
        for b in range(4):
            pltpu.async_copy(g_hbm.at[sidx.at[b]], rows.at[b], gsems[b])

        def step(k, _):
            for b in range(4):
                j = 4 * k + b
                pltpu.make_async_copy(g_hbm.at[sidx.at[j]], rows.at[b],
                                      gsems[b]).wait()
                pltpu.sync_copy(rows.at[b], acc.at[didx.at[j]], add=True)
                pltpu.async_copy(g_hbm.at[sidx.at[j + 4]], rows.at[b],
                                 gsems[b])
            return 0

        lax.fori_loop(0, NCHQ // 4 - 1, step, 0)
        for b in range(4):
            j = NCHQ - 4 + b
            pltpu.make_async_copy(g_hbm.at[sidx.at[j]], rows.at[b],
                                  gsems[b]).wait()
            pltpu.sync_copy(rows.at[b], acc.at[didx.at[j]], add=True)

    plsc.subcore_barrier()
    pltpu.sync_copy(acc.at[pl.ds(sid * ZR, ZR)],
                    out_hbm.at[cid].at[pl.ds(sid * ZR, ZR)])


@functools.partial(
    pl.kernel,
    out_type=jax.ShapeDtypeStruct((NC, NACC, W), jnp.float32),
    mesh=_mesh,
    scratch_types=[
        pltpu.VMEM((NCHQ, CHUNK), jnp.int32),
        pltpu.VMEM((NCHQ, CHUNK), jnp.int32),
        pltpu.VMEM((4, CHUNK, W), jnp.float32),
        pltpu.VMEM((ZB, W), jnp.float32),
        pltpu.VMEM_SHARED((NACC, W), jnp.float32),
        pltpu.SemaphoreType.DMA,
        pltpu.SemaphoreType.DMA,
        pltpu.SemaphoreType.DMA,
        pltpu.SemaphoreType.DMA,
    ],
)
def _scat_call(g_hbm, src_hbm, dst_hbm, out_hbm,
               sidx, didx, rows, zbuf, acc, gsem0, gsem1, gsem2, gsem3):
    _scat_body(g_hbm, src_hbm, dst_hbm, out_hbm,
               sidx, didx, rows, zbuf, acc, gsem0, gsem1, gsem2, gsem3)


def _tcA_body(deg_ref, x_ref, w1_ref, g1_ref, dinv_ref):
    deg = (deg_ref[0, :N] + deg_ref[1, :N] + 1.0)[:, None]
    dinv = lax.rsqrt(deg)
    h1 = jnp.dot(x_ref[...], w1_ref[...], preferred_element_type=jnp.float32)
    g1_ref[...] = jnp.concatenate(
        [h1 * dinv, jnp.zeros((N, W - H), jnp.float32)], axis=1)
    dinv_ref[...] = dinv


def _tcB_body(s_ref, g_ref, dinv_ref, b_ref, w_ref, out_ref):
    dinv = dinv_ref[...]
    a = ((s_ref[0, :N, :H] + s_ref[1, :N, :H] + g_ref[:, :H]) * dinv
         + b_ref[...])
    h = jnp.maximum(a, 0.0)
    g2 = jnp.dot(h, w_ref[...], preferred_element_type=jnp.float32) * dinv
    out_ref[...] = jnp.concatenate(
        [g2, jnp.zeros((N, W - H), jnp.float32)], axis=1)


def _tcC_body(s_ref, g_ref, dinv_ref, b_ref, w_ref, blin_ref, out_ref):
    a = ((s_ref[0, :N, :H] + s_ref[1, :N, :H] + g_ref[:, :H]) * dinv_ref[...]
         + b_ref[...])
    h = jnp.maximum(a, 0.0)
    out_ref[...] = jnp.dot(h, w_ref[...],
                           preferred_element_type=jnp.float32) + blin_ref[...]


def kernel(x, edge_index, W1, b1, W2, b2, Wlin, blin):
    src = edge_index[0]
    dst = edge_index[1]
    pad = E_PAD - E
    ppos = jnp.arange(pad, dtype=jnp.int32)
    src_pad = (ppos * 131) % N
    dst_pad = N + (ppos % (NACC - N))
    src_p = jnp.concatenate([src, src_pad]).reshape(ROWS_IDX, CHUNK)
    dst_p = jnp.concatenate([dst, dst_pad]).reshape(ROWS_IDX, CHUNK)

    deg_parts = _deg_call(dst_p).reshape(NC, NACC)

    g1, dinv = pl.pallas_call(
        _tcA_body,
        out_shape=(jax.ShapeDtypeStruct((N, W), jnp.float32),
                   jax.ShapeDtypeStruct((N, 1), jnp.float32)),
    )(deg_parts, x, W1)

    s1 = _scat_call(g1, src_p, dst_p)

    g2 = pl.pallas_call(
        _tcB_body,
        out_shape=jax.ShapeDtypeStruct((N, W), jnp.float32),
    )(s1, g1, dinv, b1.reshape(1, H), W2)

    s2 = _scat_call(g2, src_p, dst_p)

    logits = pl.pallas_call(
        _tcC_body,
        out_shape=jax.ShapeDtypeStruct((N, O), jnp.float32),
    )(s2, g2, dinv, b2.reshape(1, H), Wlin, blin.reshape(1, O))
    return logits


# speedup vs baseline: 1.0363x; 1.0136x over previous
import functools

import jax
import jax.numpy as jnp
from jax import lax
from jax.experimental import pallas as pl
from jax.experimental.pallas import tpu as pltpu
from jax.experimental.pallas import tpu_sc as plsc

N = 10000
E = 320000
D = 128
H = 64
O = 2

NC = 2
NS = 16
NW = NC * NS
CHUNK = 64
EPT = 10240
NCH = EPT // CHUNK
NCHQ = NCH // 4
E_PAD = NW * EPT
ROWS_IDX = E_PAD // CHUNK
NACC = 10240
ZR = NACC // NS
ZB = 40
W = 128

_mesh = plsc.VectorSubcoreMesh(core_axis_name="c", subcore_axis_name="s")


def _deg_body(dst_hbm, out_hbm, didx, ones_v, zbuf, acc, sem):
    cid = lax.axis_index("c")
    sid = lax.axis_index("s")
    w = cid * NS + sid
    pltpu.sync_copy(dst_hbm.at[pl.ds(w * NCH, NCH)], didx)
    for j in range(ZR // 16):
        zbuf[pl.ds(j * 16, 16)] = jnp.zeros((16,), jnp.float32)
    for j in range(CHUNK // 16):
        ones_v[pl.ds(j * 16, 16)] = jnp.ones((16,), jnp.float32)
    pltpu.sync_copy(zbuf, acc.at[pl.ds(sid * ZR, ZR)])
    plsc.subcore_barrier()

    def step(j, _):
        pltpu.sync_copy(ones_v, acc.at[didx.at[j]], add=True)
        return 0

    lax.fori_loop(0, NCH, step, 0)
    plsc.subcore_barrier()
    pltpu.sync_copy(acc.at[pl.ds(sid * ZR, ZR)],
                    out_hbm.at[pl.ds(cid * NACC + sid * ZR, ZR)])


@functools.partial(
    pl.kernel,
    out_type=jax.ShapeDtypeStruct((NC * NACC,), jnp.float32),
    mesh=_mesh,
    scratch_types=[
        pltpu.VMEM((NCH, CHUNK), jnp.int32),
        pltpu.VMEM((CHUNK,), jnp.float32),
        pltpu.VMEM((ZR,), jnp.float32),
        pltpu.VMEM_SHARED((NACC,), jnp.float32),
        pltpu.SemaphoreType.DMA,
    ],
)
def _deg_call(dst_hbm, out_hbm, didx, ones_v, zbuf, acc, sem):
    _deg_body(dst_hbm, out_hbm, didx, ones_v, zbuf, acc, sem)


def _scat_body(g_hbm, src_hbm, dst_hbm, out_hbm,
               sidx, didx, rows, zbuf, acc, gsem0, gsem1, gsem2, gsem3):
    cid = lax.axis_index("c")
    sid = lax.axis_index("s")
    w = cid * NS + sid
    gsems = (gsem0, gsem1, gsem2, gsem3)
    pltpu.sync_copy(src_hbm.at[pl.ds(w * NCH, NCHQ)], sidx)
    pltpu.sync_copy(dst_hbm.at[pl.ds(w * NCH, NCHQ)], didx)
    for b in range(4):
        pltpu.async_copy(g_hbm.at[sidx.at[b]], rows.at[b], gsems[b])
    for r in range(ZB):
        for c in range(W // 16):
            zbuf[r, pl.ds(c * 16, 16)] = jnp.zeros((16,), jnp.float32)

    def zstep(t, _):
        pltpu.sync_copy(zbuf, acc.at[pl.ds(sid * ZR + t * ZB, ZB)])
        return 0

    lax.fori_loop(0, ZR // ZB, zstep, 0)
    plsc.subcore_barrier()

    for quarter in range(4):
        if quarter > 0:
            base = w * NCH + quarter * NCHQ
            pltpu.sync_copy(src_hbm.at[pl.ds(base, NCHQ)], sidx)
            pltpu.sync_copy(dst_hbm.at[pl.ds(base, NCHQ)], didx)
            for b in range(4):
                pltpu.async_copy(g_hbm.at[sidx.at[b]], rows.at[b], gsems[b])

        def step(k, _):
            for b in range(4):
                j = 4 * k + b
                pltpu.make_async_copy(g_hbm.at[sidx.at[j]], rows.at[b],
                                      gsems[b]).wait()
                pltpu.sync_copy(rows.at[b], acc.at[didx.at[j]], add=True)
                pltpu.async_copy(g_hbm.at[sidx.at[j + 4]], rows.at[b],
                                 gsems[b])
            return 0

        lax.fori_loop(0, NCHQ // 4 - 1, step, 0)
        for b in range(4):
            j = NCHQ - 4 + b
            pltpu.make_async_copy(g_hbm.at[sidx.at[j]], rows.at[b],
                                  gsems[b]).wait()
            pltpu.sync_copy(rows.at[b], acc.at[didx.at[j]], add=True)

    plsc.subcore_barrier()
    pltpu.sync_copy(acc.at[pl.ds(sid * ZR, ZR)],
                    out_hbm.at[cid].at[pl.ds(sid * ZR, ZR)])


@functools.partial(
    pl.kernel,
    out_type=jax.ShapeDtypeStruct((NC, NACC, W), jnp.float32),
    mesh=_mesh,
    scratch_types=[
        pltpu.VMEM((NCHQ, CHUNK), jnp.int32),
        pltpu.VMEM((NCHQ, CHUNK), jnp.int32),
        pltpu.VMEM((4, CHUNK, W), jnp.float32),
        pltpu.VMEM((ZB, W), jnp.float32),
        pltpu.VMEM_SHARED((NACC, W), jnp.float32),
        pltpu.SemaphoreType.DMA,
        pltpu.SemaphoreType.DMA,
        pltpu.SemaphoreType.DMA,
        pltpu.SemaphoreType.DMA,
    ],
)
def _scat_call(g_hbm, src_hbm, dst_hbm, out_hbm,
               sidx, didx, rows, zbuf, acc, gsem0, gsem1, gsem2, gsem3):
    _scat_body(g_hbm, src_hbm, dst_hbm, out_hbm,
               sidx, didx, rows, zbuf, acc, gsem0, gsem1, gsem2, gsem3)


def _tcA_body(deg_ref, x_ref, w1_ref, g1_ref, dinv_ref):
    deg = (deg_ref[0, :N] + deg_ref[1, :N] + 1.0)[:, None]
    dinv = lax.rsqrt(deg)
    h1 = jnp.dot(x_ref[...], w1_ref[...], preferred_element_type=jnp.float32)
    g1_ref[...] = jnp.concatenate(
        [h1 * dinv, jnp.zeros((N, W - H), jnp.float32)], axis=1)
    dinv_ref[...] = dinv


def _tcB_body(s_ref, g_ref, dinv_ref, b_ref, w_ref, out_ref):
    dinv = dinv_ref[...]
    a = ((s_ref[0, :N, :H] + s_ref[1, :N, :H] + g_ref[:, :H]) * dinv
         + b_ref[...])
    h = jnp.maximum(a, 0.0)
    g2 = jnp.dot(h, w_ref[...], preferred_element_type=jnp.float32) * dinv
    out_ref[...] = jnp.concatenate(
        [g2, jnp.zeros((N, W - H), jnp.float32)], axis=1)


def _tcC_body(s_ref, g_ref, dinv_ref, b_ref, w_ref, blin_ref, out_ref):
    a = ((s_ref[0, :N, :H] + s_ref[1, :N, :H] + g_ref[:, :H]) * dinv_ref[...]
         + b_ref[...])
    h = jnp.maximum(a, 0.0)
    out_ref[...] = jnp.dot(h, w_ref[...],
                           preferred_element_type=jnp.float32) + blin_ref[...]


def kernel(x, edge_index, W1, b1, W2, b2, Wlin, blin):
    src = edge_index[0]
    dst = edge_index[1]
    pad = E_PAD - E
    ppos = jnp.arange(pad, dtype=jnp.int32)
    src_pad = (ppos * 131) % N
    dst_pad = N + (ppos % (NACC - N))
    src_p = jnp.concatenate([src, src_pad]).reshape(ROWS_IDX, CHUNK)
    dst_p = jnp.concatenate([dst, dst_pad]).reshape(ROWS_IDX, CHUNK)

    deg_parts = _deg_call(dst_p).reshape(NC, NACC)

    g1, dinv = pl.pallas_call(
        _tcA_body,
        out_shape=(jax.ShapeDtypeStruct((N, W), jnp.float32),
                   jax.ShapeDtypeStruct((N, 1), jnp.float32)),
    )(deg_parts, x, W1)

    s1 = _scat_call(g1, src_p, dst_p)

    g2 = pl.pallas_call(
        _tcB_body,
        out_shape=jax.ShapeDtypeStruct((N, W), jnp.float32),
    )(s1, g1, dinv, b1.reshape(1, H), W2)

    s2 = _scat_call(g2, src_p, dst_p)

    logits = pl.pallas_call(
        _tcC_body,
        out_shape=jax.ShapeDtypeStruct((N, O), jnp.float32),
    )(s2, g2, dinv, b2.reshape(1, H), Wlin, blin.reshape(1, O))
    return logits
